# Initial kernel scaffold; baseline (speedup 1.0000x reference)
import jax
import jax.numpy as jnp
from jax.experimental import pallas as pl


def kernel(x, edge_index, W_l, b_l, W_r):
    raise NotImplementedError("write your pallas kernel here")


# speedup vs baseline: 2.7378x; 2.7378x over previous
import functools

import jax
import jax.numpy as jnp
from jax import lax
from jax.experimental import pallas as pl
from jax.experimental.pallas import tpu as pltpu
from jax.experimental.pallas import tpu_sc as plsc

N = 10000
E = 160000
D = 256
H = 128
NPAD = 10240
EP = 163840
NC = 2
NS = 16
EPT = EP // NS
B = 128
NB = EPT // B
RPT = NPAD // NS


def _sc_body(xcat_ref, src2_ref, dst_ref, zrow_ref, zcnt_ref, ones_ref,
             agg_ref, cnt_ref,
             isrc, idst, rows, ones_v, sagg, scnt, sem):
    c = lax.axis_index("c")
    s = lax.axis_index("s")

    pltpu.sync_copy(ones_ref, ones_v)
    pltpu.sync_copy(zrow_ref, sagg.at[pl.ds(s * RPT, RPT)])
    pltpu.sync_copy(zcnt_ref, scnt.at[pl.ds(s * RPT, RPT)])
    plsc.subcore_barrier()

    def step(b, carry):
        off = s * EPT + b * B
        pltpu.sync_copy(src2_ref.at[pl.ds(c * EP + off, B)], isrc)
        pltpu.sync_copy(dst_ref.at[pl.ds(off, B)], idst)
        pltpu.async_copy(xcat_ref.at[isrc], rows, sem).wait()
        pltpu.sync_copy(rows, sagg.at[idst], add=True)
        pltpu.sync_copy(ones_v, scnt.at[idst], add=True)
        return carry
    lax.fori_loop(0, NB, step, 0)
    plsc.subcore_barrier()

    row = s * RPT
    pltpu.sync_copy(sagg.at[pl.ds(row, RPT)],
                    agg_ref.at[pl.ds(c * NPAD + row, RPT)])
    pltpu.sync_copy(scnt.at[pl.ds(row, RPT)],
                    cnt_ref.at[pl.ds(c * NPAD + row, RPT)])


_sc_aggregate = functools.partial(
    pl.kernel,
    out_type=(
        jax.ShapeDtypeStruct((NC * NPAD, H), jnp.float32),
        jax.ShapeDtypeStruct((NC * NPAD, 16), jnp.float32),
    ),
    mesh=plsc.VectorSubcoreMesh(
        core_axis_name="c", subcore_axis_name="s",
        num_cores=NC, num_subcores=NS),
    compiler_params=pltpu.CompilerParams(use_tc_tiling_on_sc=False),
    scratch_types=[
        pltpu.VMEM((B,), jnp.int32),
        pltpu.VMEM((B,), jnp.int32),
        pltpu.VMEM((B, H), jnp.float32),
        pltpu.VMEM((B, 16), jnp.float32),
        pltpu.VMEM_SHARED((NPAD, H), jnp.float32),
        pltpu.VMEM_SHARED((NPAD, 16), jnp.float32),
        pltpu.SemaphoreType.DMA,
    ],
)(_sc_body)


R = 80


def _tc_body(lo_ref, hi_ref, cnt_ref, x_ref, wl_ref, wr_ref, b_ref, o_ref):
    cnt = cnt_ref[:, 0:1]
    inv = 1.0 / jnp.maximum(cnt, 1.0)
    agg = jnp.concatenate([lo_ref[...], hi_ref[...]], axis=1) * inv
    acc = jnp.dot(agg, wl_ref[...], preferred_element_type=jnp.float32)
    acc = acc + jnp.dot(x_ref[...], wr_ref[...], preferred_element_type=jnp.float32)
    o_ref[...] = jnp.maximum(acc + b_ref[...], 0.0)


def kernel(x, edge_index, W_l, b_l, W_r):
    src = edge_index[0].astype(jnp.int32)
    dst = edge_index[1].astype(jnp.int32)
    pad = EP - E
    src_p = jnp.concatenate([src, jnp.zeros((pad,), jnp.int32)])
    dst_p = jnp.concatenate([dst, jnp.full((pad,), NPAD - 1, jnp.int32)])
    xcat = jnp.concatenate([x[:, :H], x[:, H:]], axis=0)
    src2 = jnp.concatenate([src_p, src_p + N])
    zrow = jnp.zeros((RPT, H), jnp.float32)
    zcnt = jnp.zeros((RPT, 16), jnp.float32)
    ones = jnp.ones((B, 16), jnp.float32)

    agg_cat, cnt_cat = _sc_aggregate(xcat, src2, dst_p, zrow, zcnt, ones)

    out = pl.pallas_call(
        _tc_body,
        out_shape=jax.ShapeDtypeStruct((N, D), jnp.float32),
        grid=(N // R,),
        in_specs=[
            pl.BlockSpec((R, H), lambda i: (i, 0)),
            pl.BlockSpec((R, H), lambda i: (i + NPAD // R, 0)),
            pl.BlockSpec((R, 16), lambda i: (i, 0)),
            pl.BlockSpec((R, D), lambda i: (i, 0)),
            pl.BlockSpec((D, D), lambda i: (0, 0)),
            pl.BlockSpec((D, D), lambda i: (0, 0)),
            pl.BlockSpec((1, D), lambda i: (0, 0)),
        ],
        out_specs=pl.BlockSpec((R, D), lambda i: (i, 0)),
    )(agg_cat, agg_cat, cnt_cat, x, W_l.T, W_r.T, b_l.reshape(1, D))
    return out


# speedup vs baseline: 3.4100x; 1.2455x over previous
import functools

import jax
import jax.numpy as jnp
from jax import lax
from jax.experimental import pallas as pl
from jax.experimental.pallas import tpu as pltpu
from jax.experimental.pallas import tpu_sc as plsc

N = 10000
E = 160000
D = 256
H = 128
NPAD = 10240
EP = 163840
NC = 2
NS = 16
EPT = EP // NS
B = 128
NB = EPT // B
K = 8
NCH = NB // K
RPT = NPAD // NS


def _sc_body(xcat_ref, src4_ref, dst3_ref, zrow_ref, zcnt_ref, ones_ref,
             agg_ref, cnt_ref,
             src_ch, dst_ch, rows_a, rows_b, ones_v, sagg, scnt,
             sem_a, sem_b):
    c = lax.axis_index("c")
    s = lax.axis_index("s")

    pltpu.sync_copy(ones_ref, ones_v)
    pltpu.sync_copy(zrow_ref, sagg.at[pl.ds(s * RPT, RPT)])
    pltpu.sync_copy(zcnt_ref, scnt.at[pl.ds(s * RPT, RPT)])
    plsc.subcore_barrier()

    rows = (rows_a, rows_b)
    sems = (sem_a, sem_b)

    def gather_start(b, buf, sem):
        pltpu.async_copy(xcat_ref.at[src_ch.at[b]], buf, sem)

    def gather_wait(buf, sem):
        pltpu.make_async_copy(xcat_ref.at[src_ch.at[0]], buf, sem).wait()

    def chunk(ch, carry):
        pltpu.sync_copy(src4_ref.at[c, s].at[pl.ds(ch * K, K)], src_ch)
        pltpu.sync_copy(dst3_ref.at[s].at[pl.ds(ch * K, K)], dst_ch)
        gather_start(0, rows[0], sems[0])
        for b in range(K):
            buf, sem = rows[b % 2], sems[b % 2]
            if b + 1 < K:
                gather_start(b + 1, rows[(b + 1) % 2], sems[(b + 1) % 2])
            gather_wait(buf, sem)
            idx = dst_ch.at[b]
            pltpu.sync_copy(buf, sagg.at[idx], add=True)
            pltpu.sync_copy(ones_v, scnt.at[idx], add=True)
        return carry
    lax.fori_loop(0, NCH, chunk, 0)
    plsc.subcore_barrier()

    row = s * RPT
    pltpu.sync_copy(sagg.at[pl.ds(row, RPT)],
                    agg_ref.at[pl.ds(c * NPAD + row, RPT)])
    pltpu.sync_copy(scnt.at[pl.ds(row, RPT)],
                    cnt_ref.at[pl.ds(c * NPAD + row, RPT)])


_sc_aggregate = functools.partial(
    pl.kernel,
    out_type=(
        jax.ShapeDtypeStruct((NC * NPAD, H), jnp.float32),
        jax.ShapeDtypeStruct((NC * NPAD, 16), jnp.float32),
    ),
    mesh=plsc.VectorSubcoreMesh(
        core_axis_name="c", subcore_axis_name="s",
        num_cores=NC, num_subcores=NS),
    compiler_params=pltpu.CompilerParams(use_tc_tiling_on_sc=False),
    scratch_types=[
        pltpu.VMEM((K, B), jnp.int32),
        pltpu.VMEM((K, B), jnp.int32),
        pltpu.VMEM((B, H), jnp.float32),
        pltpu.VMEM((B, H), jnp.float32),
        pltpu.VMEM((B, 16), jnp.float32),
        pltpu.VMEM_SHARED((NPAD, H), jnp.float32),
        pltpu.VMEM_SHARED((NPAD, 16), jnp.float32),
        pltpu.SemaphoreType.DMA,
        pltpu.SemaphoreType.DMA,
    ],
)(_sc_body)


R = 80


def _tc_body(lo_ref, hi_ref, cnt_ref, x_ref, wl_ref, wr_ref, b_ref, o_ref):
    cnt = cnt_ref[:, 0:1]
    inv = 1.0 / jnp.maximum(cnt, 1.0)
    agg = jnp.concatenate([lo_ref[...], hi_ref[...]], axis=1) * inv
    acc = jnp.dot(agg, wl_ref[...], preferred_element_type=jnp.float32)
    acc = acc + jnp.dot(x_ref[...], wr_ref[...], preferred_element_type=jnp.float32)
    o_ref[...] = jnp.maximum(acc + b_ref[...], 0.0)


def kernel(x, edge_index, W_l, b_l, W_r):
    src = edge_index[0].astype(jnp.int32)
    dst = edge_index[1].astype(jnp.int32)
    pad = EP - E
    src_p = jnp.concatenate([src, jnp.zeros((pad,), jnp.int32)])
    dst_p = jnp.concatenate([dst, jnp.full((pad,), NPAD - 1, jnp.int32)])
    src3 = src_p.reshape(NS, NB, B)
    dst3 = dst_p.reshape(NS, NB, B)
    xcat = jnp.concatenate([x[:, :H], x[:, H:]], axis=0)
    src4 = jnp.stack([src3, src3 + N])
    zrow = jnp.zeros((RPT, H), jnp.float32)
    zcnt = jnp.zeros((RPT, 16), jnp.float32)
    ones = jnp.ones((B, 16), jnp.float32)

    agg_cat, cnt_cat = _sc_aggregate(xcat, src4, dst3, zrow, zcnt, ones)

    out = pl.pallas_call(
        _tc_body,
        out_shape=jax.ShapeDtypeStruct((N, D), jnp.float32),
        grid=(N // R,),
        in_specs=[
            pl.BlockSpec((R, H), lambda i: (i, 0)),
            pl.BlockSpec((R, H), lambda i: (i + NPAD // R, 0)),
            pl.BlockSpec((R, 16), lambda i: (i, 0)),
            pl.BlockSpec((R, D), lambda i: (i, 0)),
            pl.BlockSpec((D, D), lambda i: (0, 0)),
            pl.BlockSpec((D, D), lambda i: (0, 0)),
            pl.BlockSpec((1, D), lambda i: (0, 0)),
        ],
        out_specs=pl.BlockSpec((R, D), lambda i: (i, 0)),
    )(agg_cat, agg_cat, cnt_cat, x, W_l.T, W_r.T, b_l.reshape(1, D))
    return out
